# Initial kernel scaffold; baseline (speedup 1.0000x reference)
import jax
import jax.numpy as jnp
from jax.experimental import pallas as pl


def kernel(sites, bonds, idx1, idx2, idx2_oh, W_self, W_pool, b_eq, att_W, att_b, W1, b1, W2, b2):
    raise NotImplementedError("write your pallas kernel here")


# speedup vs baseline: 19.3615x; 19.3615x over previous
import jax
import jax.numpy as jnp
from jax.experimental import pallas as pl
from jax.experimental.pallas import tpu as pltpu

B, N, E = 8, 128, 512
IN_F, HID_F, OUT_F, MSG_F, BOND_F = 64, 128, 64, 64, 16


def _leaky(x):
    return jnp.where(x >= 0, x, 0.01 * x)


def _fused_kernel(idx1_ref, sites_ref, bonds_ref, idx2_oh_ref,
                  W_self_ref, W_pool_ref, b_eq_ref, att_W_ref, att_b_ref,
                  W1_ref, b1_ref, W2_ref, b2_ref, out_ref):
    sites = sites_ref[0]
    bonds = bonds_ref[0]
    oh2 = idx2_oh_ref[...]

    idx1 = idx1_ref[...].reshape(E, 1)
    iota_n = jax.lax.broadcasted_iota(jnp.int32, (E, N), 1)
    oh1 = (iota_n == idx1).astype(jnp.float32)

    s_s = jnp.dot(oh1, sites, preferred_element_type=jnp.float32)
    s_r = jnp.dot(oh2, sites, preferred_element_type=jnp.float32)

    W_eff = W_self_ref[...] + W_pool_ref[...] * (1.0 / N)
    lat = (jnp.dot(s_s, W_eff[:IN_F], preferred_element_type=jnp.float32)
           + jnp.dot(s_r, W_eff[IN_F:2 * IN_F], preferred_element_type=jnp.float32)
           + jnp.dot(bonds, W_eff[2 * IN_F:], preferred_element_type=jnp.float32)
           + b_eq_ref[...])
    lat = _leaky(lat)

    logits = jnp.sum(lat * att_W_ref[...].T, axis=1, keepdims=True) + att_b_ref[...]
    lat = lat * jax.nn.sigmoid(logits)

    msg = jnp.dot(oh2.T, lat, preferred_element_type=jnp.float32)

    v = _leaky(jnp.dot(sites, W1_ref[:IN_F], preferred_element_type=jnp.float32)
               + jnp.dot(msg, W1_ref[IN_F:], preferred_element_type=jnp.float32)
               + b1_ref[...])
    v = _leaky(jnp.dot(v, W2_ref[...], preferred_element_type=jnp.float32)
               + b2_ref[...])
    out_ref[0] = sites + v


def kernel(sites, bonds, idx1, idx2, idx2_oh, W_self, W_pool, b_eq, att_W, att_b, W1, b1, W2, b2):
    C = 2 * IN_F + BOND_F
    idx1_2d = idx1.reshape(1, E)
    b_eq_2d = b_eq.reshape(1, MSG_F)
    att_b_2d = att_b.reshape(1, 1)
    b1_2d = b1.reshape(1, HID_F)
    b2_2d = b2.reshape(1, OUT_F)

    batch_block = lambda i: (i, 0, 0)
    fixed2 = lambda i: (0, 0)

    sites_out = pl.pallas_call(
        _fused_kernel,
        grid=(B,),
        in_specs=[
            pl.BlockSpec((1, E), fixed2),
            pl.BlockSpec((1, N, IN_F), batch_block),
            pl.BlockSpec((1, E, BOND_F), batch_block),
            pl.BlockSpec((E, N), fixed2),
            pl.BlockSpec((C, MSG_F), fixed2),
            pl.BlockSpec((C, MSG_F), fixed2),
            pl.BlockSpec((1, MSG_F), fixed2),
            pl.BlockSpec((MSG_F, 1), fixed2),
            pl.BlockSpec((1, 1), fixed2),
            pl.BlockSpec((IN_F + MSG_F, HID_F), fixed2),
            pl.BlockSpec((1, HID_F), fixed2),
            pl.BlockSpec((HID_F, OUT_F), fixed2),
            pl.BlockSpec((1, OUT_F), fixed2),
        ],
        out_specs=pl.BlockSpec((1, N, OUT_F), batch_block),
        out_shape=jax.ShapeDtypeStruct((B, N, OUT_F), jnp.float32),
    )(idx1_2d, sites, bonds, idx2_oh, W_self, W_pool, b_eq_2d,
      att_W, att_b_2d, W1, b1_2d, W2, b2_2d)

    return (sites_out, bonds)
